# Initial kernel scaffold; baseline (speedup 1.0000x reference)
import functools

import jax
import jax.numpy as jnp
from jax import lax
from jax.experimental import pallas as pl
from jax.experimental.pallas import tpu as pltpu
from jax.experimental.pallas import tpu_sc as plsc

N_NODES = 10000
N_EDGES = 320000
NC = 2
NS = 16
NW = NC * NS
EPT = N_EDGES // NW
CHUNK = 125
NCHUNK = EPT // CHUNK
RPT = N_NODES // NS

_MESH = plsc.VectorSubcoreMesh(core_axis_name="c", subcore_axis_name="s")


def _degree(dst3, ones16, zeros16):
  @functools.partial(
      pl.kernel,
      out_type=jax.ShapeDtypeStruct((NC, N_NODES, 16), jnp.float32),
      mesh=_MESH,
      scratch_types=[
          pltpu.VMEM((NCHUNK, CHUNK), jnp.int32),
          pltpu.VMEM((CHUNK, 16), jnp.float32),
          pltpu.VMEM_SHARED((N_NODES, 16), jnp.float32),
      ],
  )
  def run(dst_hbm, ones_hbm, zeros_hbm, out_hbm, dst_v, ones_v, acc):
    c = lax.axis_index("c")
    s = lax.axis_index("s")
    wid = s * NC + c
    pltpu.sync_copy(dst_hbm.at[wid], dst_v)
    pltpu.sync_copy(ones_hbm, ones_v)
    base = s * RPT
    pltpu.sync_copy(zeros_hbm, acc.at[pl.ds(base, RPT)])
    plsc.subcore_barrier()

    def body(j, carry):
      pltpu.sync_copy(ones_v, acc.at[dst_v.at[j]], add=True)
      return carry

    lax.fori_loop(0, NCHUNK, body, 0)
    plsc.subcore_barrier()
    pltpu.sync_copy(acc.at[pl.ds(base, RPT)], out_hbm.at[c].at[pl.ds(base, RPT)])

  return run(dst3, ones16, zeros16)


def _aggregate(mp, src3, dst3, zeros, feat):
  @functools.partial(
      pl.kernel,
      out_type=jax.ShapeDtypeStruct((NC, N_NODES, feat), jnp.float32),
      mesh=_MESH,
      scratch_types=[
          pltpu.VMEM((NCHUNK, CHUNK), jnp.int32),
          pltpu.VMEM((NCHUNK, CHUNK), jnp.int32),
          pltpu.VMEM((CHUNK, feat), jnp.float32),
          pltpu.VMEM((CHUNK, feat), jnp.float32),
          pltpu.VMEM_SHARED((N_NODES, feat), jnp.float32),
          pltpu.SemaphoreType.DMA,
      ],
  )
  def run(mp_hbm, src_hbm, dst_hbm, zeros_hbm, out_hbm,
          src_v, dst_v, buf0, buf1, acc, sem):
    c = lax.axis_index("c")
    s = lax.axis_index("s")
    wid = s * NC + c
    pltpu.sync_copy(src_hbm.at[wid], src_v)
    pltpu.sync_copy(dst_hbm.at[wid], dst_v)
    base = s * RPT

    @pl.when(c == 0)
    def _():
      pltpu.sync_copy(mp_hbm.at[pl.ds(base, RPT)], acc.at[pl.ds(base, RPT)])

    @pl.when(c != 0)
    def _():
      pltpu.sync_copy(zeros_hbm, acc.at[pl.ds(base, RPT)])

    plsc.subcore_barrier()

    bufs = (buf0, buf1)
    pltpu.async_copy(mp_hbm.at[src_v.at[0]], buf0, sem)

    def outer(i, carry):
      j0 = i * 2
      for b in range(2):
        j = j0 + b
        pltpu.make_async_copy(mp_hbm.at[src_v.at[j]], bufs[b], sem).wait()

        @pl.when(j + 1 < NCHUNK)
        def _():
          pltpu.async_copy(mp_hbm.at[src_v.at[j + 1]], bufs[1 - b], sem)

        pltpu.sync_copy(bufs[b], acc.at[dst_v.at[j]], add=True)
      return carry

    lax.fori_loop(0, NCHUNK // 2, outer, 0)
    plsc.subcore_barrier()
    pltpu.sync_copy(acc.at[pl.ds(base, RPT)], out_hbm.at[c].at[pl.ds(base, RPT)])

  return run(mp, src3, dst3, zeros)


def _tc_first(features, W1, deg):
  def body(f_ref, w_ref, d_ref, mp_ref, dis_ref):
    dis = lax.rsqrt(d_ref[0] + d_ref[1] + 1.0)
    dis_ref[...] = dis
    m = jnp.dot(f_ref[...], w_ref[...],
                preferred_element_type=jnp.float32,
                precision=lax.Precision.HIGHEST)
    mp_ref[...] = m * dis[:, 0:1]

  return pl.pallas_call(
      body,
      out_shape=(jax.ShapeDtypeStruct((N_NODES, 128), jnp.float32),
                 jax.ShapeDtypeStruct((N_NODES, 16), jnp.float32)),
  )(features, W1, deg)


def _tc_mid(acc, dis16, b, W, feat_out):
  def body(a_ref, dis_ref, b_ref, w_ref, o_ref):
    dis = dis_ref[...][:, 0:1]
    x = jnp.maximum(dis * (a_ref[0] + a_ref[1]) + b_ref[...], 0.0)
    o_ref[...] = jnp.dot(x, w_ref[...],
                         preferred_element_type=jnp.float32,
                         precision=lax.Precision.HIGHEST) * dis

  return pl.pallas_call(
      body,
      out_shape=jax.ShapeDtypeStruct((N_NODES, feat_out), jnp.float32),
  )(acc, dis16, b, W)


def _tc_final(acc, dis16, b):
  def body(a_ref, dis_ref, b_ref, x_ref, ls_ref):
    dis = dis_ref[...][:, 0:1]
    x = dis * (a_ref[0] + a_ref[1]) + b_ref[...]
    x_ref[...] = x
    mx = jnp.max(x, axis=-1, keepdims=True)
    lse = jnp.log(jnp.sum(jnp.exp(x - mx), axis=-1, keepdims=True)) + mx
    ls_ref[...] = x - lse

  return pl.pallas_call(
      body,
      out_shape=(jax.ShapeDtypeStruct((N_NODES, 64), jnp.float32),
                 jax.ShapeDtypeStruct((N_NODES, 64), jnp.float32)),
  )(acc, dis16, b)


def kernel(features, edge_index, W1, b1, W2, b2, W3, b3):
  ei = edge_index.astype(jnp.int32)
  src3 = ei[0].reshape(NW, NCHUNK, CHUNK)
  dst3 = ei[1].reshape(NW, NCHUNK, CHUNK)
  ones16 = jnp.ones((CHUNK, 16), jnp.float32)
  zeros16 = jnp.zeros((RPT, 16), jnp.float32)
  zeros128 = jnp.zeros((RPT, 128), jnp.float32)
  zeros64 = jnp.zeros((RPT, 64), jnp.float32)
  b1r = b1.reshape(1, -1)
  b2r = b2.reshape(1, -1)
  b3r = b3.reshape(1, -1)

  deg = _degree(dst3, ones16, zeros16)
  mp1, dis16 = _tc_first(features, W1, deg)
  acc1 = _aggregate(mp1, src3, dst3, zeros128, 128)
  mp2 = _tc_mid(acc1, dis16, b1r, W2, 128)
  acc2 = _aggregate(mp2, src3, dst3, zeros128, 128)
  mp3 = _tc_mid(acc2, dis16, b2r, W3, 64)
  acc3 = _aggregate(mp3, src3, dst3, zeros64, 64)
  return _tc_final(acc3, dis16, b3r)


# speedup vs baseline: 5.2220x; 5.2220x over previous
import functools

import jax
import jax.numpy as jnp
from jax import lax
from jax.experimental import pallas as pl
from jax.experimental.pallas import tpu as pltpu
from jax.experimental.pallas import tpu_sc as plsc

N_NODES = 10000
N_PAD = 10240
N_EDGES = 320000
FEAT = 128
NC = 2
NS = 16
NW = NC * NS
CHUNK = 128
NCHP = 80
EPT = NCHP * CHUNK
E_PAD = NW * EPT
GARBAGE = N_PAD - 1
RPT = N_PAD // NS
TCB = 1280
TCG = N_PAD // TCB

_MESH = plsc.VectorSubcoreMesh(core_axis_name="c", subcore_axis_name="s")


def _aggregate(mp, src3, dst3, zeros):
  @functools.partial(
      pl.kernel,
      out_type=jax.ShapeDtypeStruct((NC, N_PAD, FEAT), jnp.float32),
      mesh=_MESH,
      scratch_types=[
          pltpu.VMEM((2, CHUNK), jnp.int32),
          pltpu.VMEM((2, CHUNK), jnp.int32),
          pltpu.VMEM((CHUNK, FEAT), jnp.float32),
          pltpu.VMEM((CHUNK, FEAT), jnp.float32),
          pltpu.VMEM_SHARED((N_PAD, FEAT), jnp.float32),
          pltpu.SemaphoreType.DMA,
          pltpu.SemaphoreType.DMA,
      ],
  )
  def run(mp_hbm, src_hbm, dst_hbm, zeros_hbm, out_hbm,
          srcr, dstr, buf0, buf1, acc, sem, semp):
    c = lax.axis_index("c")
    s = lax.axis_index("s")
    wid = s * NC + c
    base = s * RPT

    @pl.when(c == 0)
    def _():
      pltpu.sync_copy(mp_hbm.at[pl.ds(base, RPT)], acc.at[pl.ds(base, RPT)])

    @pl.when(c != 0)
    def _():
      pltpu.sync_copy(zeros_hbm, acc.at[pl.ds(base, RPT)])

    plsc.subcore_barrier()

    bufs = (buf0, buf1)
    pltpu.sync_copy(src_hbm.at[wid].at[0], srcr.at[0])
    pltpu.sync_copy(dst_hbm.at[wid].at[0], dstr.at[0])
    pltpu.async_copy(mp_hbm.at[srcr.at[0]], buf0, sem)
    pltpu.async_copy(src_hbm.at[wid].at[1], srcr.at[1], semp)
    pltpu.async_copy(dst_hbm.at[wid].at[1], dstr.at[1], semp)

    def outer(i, carry):
      j0 = i * 2
      for b in range(2):
        j = j0 + b

        @pl.when(j + 1 < NCHP)
        def _():
          pltpu.make_async_copy(src_hbm.at[wid].at[j + 1], srcr.at[1 - b], semp).wait()
          pltpu.make_async_copy(dst_hbm.at[wid].at[j + 1], dstr.at[1 - b], semp).wait()

        pltpu.make_async_copy(mp_hbm.at[srcr.at[b]], bufs[b], sem).wait()

        @pl.when(j + 1 < NCHP)
        def _():
          pltpu.async_copy(mp_hbm.at[srcr.at[1 - b]], bufs[1 - b], sem)

        pltpu.sync_copy(bufs[b], acc.at[dstr.at[b]], add=True)

        @pl.when(j + 2 < NCHP)
        def _():
          pltpu.async_copy(src_hbm.at[wid].at[j + 2], srcr.at[b], semp)
          pltpu.async_copy(dst_hbm.at[wid].at[j + 2], dstr.at[b], semp)
      return carry

    lax.fori_loop(0, NCHP // 2, outer, 0)
    plsc.subcore_barrier()
    pltpu.sync_copy(acc.at[pl.ds(base, RPT)], out_hbm.at[c].at[pl.ds(base, RPT)])

  return run(mp, src3, dst3, zeros)


def _tc_first(features, W1, deg):
  def body(f_ref, w_ref, d_ref, mp_ref, dis_ref):
    dis = lax.rsqrt(d_ref[0][:, :16] + d_ref[1][:, :16])
    dis_ref[...] = dis
    m = jnp.dot(f_ref[...], w_ref[...],
                preferred_element_type=jnp.float32,
                precision=lax.Precision.HIGHEST)
    mp_ref[...] = m * dis[:, 0:1]

  return pl.pallas_call(
      body,
      grid=(TCG,),
      in_specs=[
          pl.BlockSpec((TCB, FEAT), lambda i: (i, 0)),
          pl.BlockSpec((FEAT, FEAT), lambda i: (0, 0)),
          pl.BlockSpec((NC, TCB, FEAT), lambda i: (0, i, 0)),
      ],
      out_specs=(pl.BlockSpec((TCB, FEAT), lambda i: (i, 0)),
                 pl.BlockSpec((TCB, 16), lambda i: (i, 0))),
      out_shape=(jax.ShapeDtypeStruct((N_PAD, FEAT), jnp.float32),
                 jax.ShapeDtypeStruct((N_PAD, 16), jnp.float32)),
  )(features, W1, deg)


def _tc_mid(acc, dis16, b, W, feat_out):
  def body(a_ref, dis_ref, b_ref, w_ref, o_ref):
    dis = dis_ref[...][:, 0:1]
    x = jnp.maximum(dis * (a_ref[0] + a_ref[1]) + b_ref[...], 0.0)
    m = jnp.dot(x, w_ref[...],
                preferred_element_type=jnp.float32,
                precision=lax.Precision.HIGHEST) * dis
    if feat_out < FEAT:
      m = jnp.concatenate(
          [m, jnp.zeros((TCB, FEAT - feat_out), jnp.float32)], axis=1)
    o_ref[...] = m

  return pl.pallas_call(
      body,
      grid=(TCG,),
      in_specs=[
          pl.BlockSpec((NC, TCB, FEAT), lambda i: (0, i, 0)),
          pl.BlockSpec((TCB, 16), lambda i: (i, 0)),
          pl.BlockSpec((1, FEAT), lambda i: (0, 0)),
          pl.BlockSpec((FEAT, feat_out), lambda i: (0, 0)),
      ],
      out_specs=pl.BlockSpec((TCB, FEAT), lambda i: (i, 0)),
      out_shape=jax.ShapeDtypeStruct((N_PAD, FEAT), jnp.float32),
  )(acc, dis16, b, W)


def _tc_final(acc, dis16, b):
  def body(a_ref, dis_ref, b_ref, x_ref, ls_ref):
    dis = dis_ref[...][:, 0:1]
    x = dis * (a_ref[0] + a_ref[1])[:, :64] + b_ref[...]
    x_ref[...] = x
    mx = jnp.max(x, axis=-1, keepdims=True)
    lse = jnp.log(jnp.sum(jnp.exp(x - mx), axis=-1, keepdims=True)) + mx
    ls_ref[...] = x - lse

  return pl.pallas_call(
      body,
      grid=(TCG,),
      in_specs=[
          pl.BlockSpec((NC, TCB, FEAT), lambda i: (0, i, 0)),
          pl.BlockSpec((TCB, 16), lambda i: (i, 0)),
          pl.BlockSpec((1, 64), lambda i: (0, 0)),
      ],
      out_specs=(pl.BlockSpec((TCB, 64), lambda i: (i, 0)),
                 pl.BlockSpec((TCB, 64), lambda i: (i, 0))),
      out_shape=(jax.ShapeDtypeStruct((N_PAD, 64), jnp.float32),
                 jax.ShapeDtypeStruct((N_PAD, 64), jnp.float32)),
  )(acc, dis16, b)


def kernel(features, edge_index, W1, b1, W2, b2, W3, b3):
  featp = jnp.pad(features, ((0, N_PAD - N_NODES), (0, 0)))
  ei = edge_index.astype(jnp.int32)
  n_fill = E_PAD - N_EDGES
  src3 = jnp.concatenate([ei[0], jnp.zeros((n_fill,), jnp.int32)]
                         ).reshape(NW, NCHP, CHUNK)
  dst3 = jnp.concatenate([ei[1], jnp.full((n_fill,), GARBAGE, jnp.int32)]
                         ).reshape(NW, NCHP, CHUNK)
  zeros128 = jnp.zeros((RPT, FEAT), jnp.float32)
  ones_nf = jnp.ones((N_PAD, FEAT), jnp.float32)
  b1r = b1.reshape(1, -1)
  b2r = b2.reshape(1, -1)
  b3r = b3.reshape(1, -1)

  dega = _aggregate(ones_nf, src3, dst3, zeros128)
  mp1, dis16 = _tc_first(featp, W1, dega)
  acc1 = _aggregate(mp1, src3, dst3, zeros128)
  mp2 = _tc_mid(acc1, dis16, b1r, W2, 128)
  acc2 = _aggregate(mp2, src3, dst3, zeros128)
  mp3 = _tc_mid(acc2, dis16, b2r, W3, 64)
  acc3 = _aggregate(mp3, src3, dst3, zeros128)
  x3, ls = _tc_final(acc3, dis16, b3r)
  return (x3[:N_NODES], ls[:N_NODES])


# speedup vs baseline: 5.8726x; 1.1246x over previous
import functools

import jax
import jax.numpy as jnp
from jax import lax
from jax.experimental import pallas as pl
from jax.experimental.pallas import tpu as pltpu
from jax.experimental.pallas import tpu_sc as plsc

N_NODES = 10000
N_PAD = 10240
N_EDGES = 320000
FEAT = 128
NC = 2
NS = 16
NW = NC * NS
CHUNK = 128
NCHP = 80
EPT = NCHP * CHUNK
E_PAD = NW * EPT
GARBAGE = N_PAD - 1
RPT = N_PAD // NS
TCB = 1280
TCG = N_PAD // TCB

_MESH = plsc.VectorSubcoreMesh(core_axis_name="c", subcore_axis_name="s")


def _aggregate(mp, src3, dst3, zeros):
  @functools.partial(
      pl.kernel,
      out_type=jax.ShapeDtypeStruct((NC, N_PAD, FEAT), jnp.float32),
      mesh=_MESH,
      scratch_types=[
          pltpu.VMEM((2, CHUNK), jnp.int32),
          pltpu.VMEM((2, CHUNK), jnp.int32),
          pltpu.VMEM((CHUNK, FEAT), jnp.float32),
          pltpu.VMEM((CHUNK, FEAT), jnp.float32),
          pltpu.VMEM_SHARED((N_PAD, FEAT), jnp.float32),
          pltpu.SemaphoreType.DMA,
          pltpu.SemaphoreType.DMA,
      ],
  )
  def run(mp_hbm, src_hbm, dst_hbm, zeros_hbm, out_hbm,
          srcr, dstr, buf0, buf1, acc, sem, semp):
    c = lax.axis_index("c")
    s = lax.axis_index("s")
    wid = s * NC + c
    base = s * RPT

    pltpu.sync_copy(zeros_hbm, acc.at[pl.ds(base, RPT)])

    plsc.subcore_barrier()

    bufs = (buf0, buf1)
    pltpu.sync_copy(src_hbm.at[wid].at[0], srcr.at[0])
    pltpu.sync_copy(dst_hbm.at[wid].at[0], dstr.at[0])
    pltpu.async_copy(mp_hbm.at[srcr.at[0]], buf0, sem)
    pltpu.async_copy(src_hbm.at[wid].at[1], srcr.at[1], semp)
    pltpu.async_copy(dst_hbm.at[wid].at[1], dstr.at[1], semp)

    def outer(i, carry):
      j0 = i * 2
      for b in range(2):
        j = j0 + b

        @pl.when(j + 1 < NCHP)
        def _():
          pltpu.make_async_copy(src_hbm.at[wid].at[j + 1], srcr.at[1 - b], semp).wait()
          pltpu.make_async_copy(dst_hbm.at[wid].at[j + 1], dstr.at[1 - b], semp).wait()

        pltpu.make_async_copy(mp_hbm.at[srcr.at[b]], bufs[b], sem).wait()

        @pl.when(j + 1 < NCHP)
        def _():
          pltpu.async_copy(mp_hbm.at[srcr.at[1 - b]], bufs[1 - b], sem)

        pltpu.sync_copy(bufs[b], acc.at[dstr.at[b]], add=True)

        @pl.when(j + 2 < NCHP)
        def _():
          pltpu.async_copy(src_hbm.at[wid].at[j + 2], srcr.at[b], semp)
          pltpu.async_copy(dst_hbm.at[wid].at[j + 2], dstr.at[b], semp)
      return carry

    lax.fori_loop(0, NCHP // 2, outer, 0)
    plsc.subcore_barrier()
    pltpu.sync_copy(acc.at[pl.ds(base, RPT)], out_hbm.at[c].at[pl.ds(base, RPT)])

  return run(mp, src3, dst3, zeros)


def _tc_first(features, W1, deg):
  def body(f_ref, w_ref, d_ref, mp_ref, dis_ref):
    dis = lax.rsqrt(d_ref[0][:, :16] + d_ref[1][:, :16] + 1.0)
    dis_ref[...] = dis
    m = jnp.dot(f_ref[...], w_ref[...],
                preferred_element_type=jnp.float32,
                precision=lax.Precision.HIGHEST)
    mp_ref[...] = m * dis[:, 0:1]

  return pl.pallas_call(
      body,
      grid=(TCG,),
      in_specs=[
          pl.BlockSpec((TCB, FEAT), lambda i: (i, 0)),
          pl.BlockSpec((FEAT, FEAT), lambda i: (0, 0)),
          pl.BlockSpec((NC, TCB, FEAT), lambda i: (0, i, 0)),
      ],
      out_specs=(pl.BlockSpec((TCB, FEAT), lambda i: (i, 0)),
                 pl.BlockSpec((TCB, 16), lambda i: (i, 0))),
      out_shape=(jax.ShapeDtypeStruct((N_PAD, FEAT), jnp.float32),
                 jax.ShapeDtypeStruct((N_PAD, 16), jnp.float32)),
  )(features, W1, deg)


def _tc_mid(acc, mp, dis16, b, W, feat_out):
  def body(a_ref, mp_ref, dis_ref, b_ref, w_ref, o_ref):
    dis = dis_ref[...][:, 0:1]
    x = jnp.maximum(dis * (a_ref[0] + a_ref[1] + mp_ref[...]) + b_ref[...], 0.0)
    m = jnp.dot(x, w_ref[...],
                preferred_element_type=jnp.float32,
                precision=lax.Precision.HIGHEST) * dis
    if feat_out < FEAT:
      m = jnp.concatenate(
          [m, jnp.zeros((TCB, FEAT - feat_out), jnp.float32)], axis=1)
    o_ref[...] = m

  return pl.pallas_call(
      body,
      grid=(TCG,),
      in_specs=[
          pl.BlockSpec((NC, TCB, FEAT), lambda i: (0, i, 0)),
          pl.BlockSpec((TCB, FEAT), lambda i: (i, 0)),
          pl.BlockSpec((TCB, 16), lambda i: (i, 0)),
          pl.BlockSpec((1, FEAT), lambda i: (0, 0)),
          pl.BlockSpec((FEAT, feat_out), lambda i: (0, 0)),
      ],
      out_specs=pl.BlockSpec((TCB, FEAT), lambda i: (i, 0)),
      out_shape=jax.ShapeDtypeStruct((N_PAD, FEAT), jnp.float32),
  )(acc, mp, dis16, b, W)


def _tc_final(acc, mp, dis16, b):
  def body(a_ref, mp_ref, dis_ref, b_ref, x_ref, ls_ref):
    dis = dis_ref[...][:, 0:1]
    x = dis * (a_ref[0] + a_ref[1] + mp_ref[...])[:, :64] + b_ref[...]
    x_ref[...] = x
    mx = jnp.max(x, axis=-1, keepdims=True)
    lse = jnp.log(jnp.sum(jnp.exp(x - mx), axis=-1, keepdims=True)) + mx
    ls_ref[...] = x - lse

  return pl.pallas_call(
      body,
      grid=(TCG,),
      in_specs=[
          pl.BlockSpec((NC, TCB, FEAT), lambda i: (0, i, 0)),
          pl.BlockSpec((TCB, FEAT), lambda i: (i, 0)),
          pl.BlockSpec((TCB, 16), lambda i: (i, 0)),
          pl.BlockSpec((1, 64), lambda i: (0, 0)),
      ],
      out_specs=(pl.BlockSpec((TCB, 64), lambda i: (i, 0)),
                 pl.BlockSpec((TCB, 64), lambda i: (i, 0))),
      out_shape=(jax.ShapeDtypeStruct((N_PAD, 64), jnp.float32),
                 jax.ShapeDtypeStruct((N_PAD, 64), jnp.float32)),
  )(acc, mp, dis16, b)


def kernel(features, edge_index, W1, b1, W2, b2, W3, b3):
  featp = jnp.pad(features, ((0, N_PAD - N_NODES), (0, 0)))
  ei = edge_index.astype(jnp.int32)
  n_fill = E_PAD - N_EDGES
  src3 = jnp.concatenate([ei[0], jnp.zeros((n_fill,), jnp.int32)]
                         ).reshape(NW, NCHP, CHUNK)
  dst3 = jnp.concatenate([ei[1], jnp.full((n_fill,), GARBAGE, jnp.int32)]
                         ).reshape(NW, NCHP, CHUNK)
  zeros128 = jnp.zeros((RPT, FEAT), jnp.float32)
  ones_nf = jnp.ones((N_PAD, FEAT), jnp.float32)
  b1r = b1.reshape(1, -1)
  b2r = b2.reshape(1, -1)
  b3r = b3.reshape(1, -1)

  dega = _aggregate(ones_nf, src3, dst3, zeros128)
  mp1, dis16 = _tc_first(featp, W1, dega)
  acc1 = _aggregate(mp1, src3, dst3, zeros128)
  mp2 = _tc_mid(acc1, mp1, dis16, b1r, W2, 128)
  acc2 = _aggregate(mp2, src3, dst3, zeros128)
  mp3 = _tc_mid(acc2, mp2, dis16, b2r, W3, 64)
  acc3 = _aggregate(mp3, src3, dst3, zeros128)
  x3, ls = _tc_final(acc3, mp3, dis16, b3r)
  return (x3[:N_NODES], ls[:N_NODES])


# speedup vs baseline: 19.2599x; 3.2796x over previous
import functools

import jax
import jax.numpy as jnp
from jax import lax
from jax.experimental import pallas as pl
from jax.experimental.pallas import tpu as pltpu
from jax.experimental.pallas import tpu_sc as plsc

N_NODES = 10000
N_PAD = 10240
N_EDGES = 320000
FEAT = 128
NC = 2
NS = 16
NW = NC * NS
CHUNK = 128
NCHP = 80
EPT = NCHP * CHUNK
E_PAD = NW * EPT
GARBAGE = N_PAD - 1
RPT = N_PAD // NS
TCB = 1280
TCG = N_PAD // TCB

_MESH = plsc.VectorSubcoreMesh(core_axis_name="c", subcore_axis_name="s")


def _aggregate(mp, src3, dst3, zeros):
  @functools.partial(
      pl.kernel,
      out_type=jax.ShapeDtypeStruct((NC, N_PAD, FEAT), jnp.float32),
      mesh=_MESH,
      scratch_types=[
          pltpu.VMEM((2, CHUNK), jnp.int32),
          pltpu.VMEM((2, CHUNK), jnp.int32),
          pltpu.VMEM((CHUNK, FEAT), jnp.float32),
          pltpu.VMEM((CHUNK, FEAT), jnp.float32),
          pltpu.VMEM_SHARED((N_PAD, FEAT), jnp.float32),
          pltpu.SemaphoreType.DMA,
          pltpu.SemaphoreType.DMA,
      ],
  )
  def run(mp_hbm, src_hbm, dst_hbm, zeros_hbm, out_hbm,
          srcr, dstr, buf0, buf1, acc, sem, semp):
    c = lax.axis_index("c")
    s = lax.axis_index("s")
    wid = s * NC + c
    base = s * RPT

    pltpu.sync_copy(zeros_hbm, acc.at[pl.ds(base, RPT)])

    plsc.subcore_barrier()

    bufs = (buf0, buf1)
    pltpu.sync_copy(src_hbm.at[wid].at[0], srcr.at[0])
    pltpu.sync_copy(dst_hbm.at[wid].at[0], dstr.at[0])
    pltpu.async_copy(mp_hbm.at[srcr.at[0]], buf0, sem)
    pltpu.async_copy(src_hbm.at[wid].at[1], srcr.at[1], semp)
    pltpu.async_copy(dst_hbm.at[wid].at[1], dstr.at[1], semp)

    def outer(i, carry):
      j0 = i * 2
      for b in range(2):
        j = j0 + b

        @pl.when(j + 1 < NCHP)
        def _():
          pltpu.make_async_copy(src_hbm.at[wid].at[j + 1], srcr.at[1 - b], semp).wait()
          pltpu.make_async_copy(dst_hbm.at[wid].at[j + 1], dstr.at[1 - b], semp).wait()

        pltpu.make_async_copy(mp_hbm.at[srcr.at[b]], bufs[b], sem).wait()

        @pl.when(j + 1 < NCHP)
        def _():
          pltpu.async_copy(mp_hbm.at[srcr.at[1 - b]], bufs[1 - b], sem)

        pltpu.sync_copy(bufs[b], acc.at[dstr.at[b]], add=True)

        @pl.when(j + 2 < NCHP)
        def _():
          pltpu.async_copy(src_hbm.at[wid].at[j + 2], srcr.at[b], semp)
          pltpu.async_copy(dst_hbm.at[wid].at[j + 2], dstr.at[b], semp)
      return carry

    lax.fori_loop(0, NCHP // 2, outer, 0)
    plsc.subcore_barrier()
    pltpu.sync_copy(acc.at[pl.ds(base, RPT)], out_hbm.at[c].at[pl.ds(base, RPT)])

  return run(mp, src3, dst3, zeros)


def _tc_first(features, W1, deg):
  def body(f_ref, w_ref, d_ref, mp_ref, dis_ref):
    dis = lax.rsqrt(d_ref[0][:, :16] + d_ref[1][:, :16] + 1.0)
    dis_ref[...] = dis
    m = jnp.dot(f_ref[...], w_ref[...],
                preferred_element_type=jnp.float32,
                precision=lax.Precision.HIGHEST)
    mp_ref[...] = m * dis[:, 0:1]

  return pl.pallas_call(
      body,
      grid=(TCG,),
      in_specs=[
          pl.BlockSpec((TCB, FEAT), lambda i: (i, 0)),
          pl.BlockSpec((FEAT, FEAT), lambda i: (0, 0)),
          pl.BlockSpec((NC, TCB, FEAT), lambda i: (0, i, 0)),
      ],
      out_specs=(pl.BlockSpec((TCB, FEAT), lambda i: (i, 0)),
                 pl.BlockSpec((TCB, 16), lambda i: (i, 0))),
      out_shape=(jax.ShapeDtypeStruct((N_PAD, FEAT), jnp.float32),
                 jax.ShapeDtypeStruct((N_PAD, 16), jnp.float32)),
  )(features, W1, deg)


def _tc_mid(acc, mp, dis16, b, W, feat_out):
  def body(a_ref, mp_ref, dis_ref, b_ref, w_ref, o_ref):
    dis = dis_ref[...][:, 0:1]
    x = jnp.maximum(dis * (a_ref[0] + a_ref[1] + mp_ref[...]) + b_ref[...], 0.0)
    m = jnp.dot(x, w_ref[...],
                preferred_element_type=jnp.float32,
                precision=lax.Precision.HIGHEST) * dis
    if feat_out < FEAT:
      m = jnp.concatenate(
          [m, jnp.zeros((TCB, FEAT - feat_out), jnp.float32)], axis=1)
    o_ref[...] = m

  return pl.pallas_call(
      body,
      grid=(TCG,),
      in_specs=[
          pl.BlockSpec((NC, TCB, FEAT), lambda i: (0, i, 0)),
          pl.BlockSpec((TCB, FEAT), lambda i: (i, 0)),
          pl.BlockSpec((TCB, 16), lambda i: (i, 0)),
          pl.BlockSpec((1, FEAT), lambda i: (0, 0)),
          pl.BlockSpec((FEAT, feat_out), lambda i: (0, 0)),
      ],
      out_specs=pl.BlockSpec((TCB, FEAT), lambda i: (i, 0)),
      out_shape=jax.ShapeDtypeStruct((N_PAD, FEAT), jnp.float32),
  )(acc, mp, dis16, b, W)


def _tc_final(acc, mp, dis16, b):
  def body(a_ref, mp_ref, dis_ref, b_ref, x_ref, ls_ref):
    dis = dis_ref[...][:, 0:1]
    x = dis * (a_ref[0] + a_ref[1] + mp_ref[...])[:, :64] + b_ref[...]
    x_ref[...] = x
    mx = jnp.max(x, axis=-1, keepdims=True)
    lse = jnp.log(jnp.sum(jnp.exp(x - mx), axis=-1, keepdims=True)) + mx
    ls_ref[...] = x - lse

  return pl.pallas_call(
      body,
      grid=(TCG,),
      in_specs=[
          pl.BlockSpec((NC, TCB, FEAT), lambda i: (0, i, 0)),
          pl.BlockSpec((TCB, FEAT), lambda i: (i, 0)),
          pl.BlockSpec((TCB, 16), lambda i: (i, 0)),
          pl.BlockSpec((1, 64), lambda i: (0, 0)),
      ],
      out_specs=(pl.BlockSpec((TCB, 64), lambda i: (i, 0)),
                 pl.BlockSpec((TCB, 64), lambda i: (i, 0))),
      out_shape=(jax.ShapeDtypeStruct((N_PAD, 64), jnp.float32),
                 jax.ShapeDtypeStruct((N_PAD, 64), jnp.float32)),
  )(acc, mp, dis16, b)


def kernel(features, edge_index, W1, b1, W2, b2, W3, b3):
  featp = jnp.pad(features, ((0, N_PAD - N_NODES), (0, 0)))
  ei = edge_index.astype(jnp.int32)
  n_fill = E_PAD - N_EDGES
  fill = jnp.arange(n_fill, dtype=jnp.int32)
  src3 = jnp.concatenate([ei[0], fill % N_NODES]).reshape(NW, NCHP, CHUNK)
  dst3 = jnp.concatenate([ei[1], N_NODES + fill % (N_PAD - N_NODES)]
                         ).reshape(NW, NCHP, CHUNK)
  zeros128 = jnp.zeros((RPT, FEAT), jnp.float32)
  ones_nf = jnp.ones((N_PAD, FEAT), jnp.float32)
  b1r = b1.reshape(1, -1)
  b2r = b2.reshape(1, -1)
  b3r = b3.reshape(1, -1)

  dega = _aggregate(ones_nf, src3, dst3, zeros128)
  mp1, dis16 = _tc_first(featp, W1, dega)
  acc1 = _aggregate(mp1, src3, dst3, zeros128)
  mp2 = _tc_mid(acc1, mp1, dis16, b1r, W2, 128)
  acc2 = _aggregate(mp2, src3, dst3, zeros128)
  mp3 = _tc_mid(acc2, mp2, dis16, b2r, W3, 64)
  acc3 = _aggregate(mp3, src3, dst3, zeros128)
  x3, ls = _tc_final(acc3, mp3, dis16, b3r)
  return (x3[:N_NODES], ls[:N_NODES])


# speedup vs baseline: 21.3946x; 1.1108x over previous
import functools

import jax
import jax.numpy as jnp
from jax import lax
from jax.experimental import pallas as pl
from jax.experimental.pallas import tpu as pltpu
from jax.experimental.pallas import tpu_sc as plsc

N_NODES = 10000
N_PAD = 10240
N_EDGES = 320000
FEAT = 128
NC = 2
NS = 16
NW = NC * NS
CHUNK = 128
NCHP = 80
EPT = NCHP * CHUNK
E_PAD = NW * EPT
GARBAGE = N_PAD - 1
RPT = N_PAD // NS
TCB = 1280
TCG = N_PAD // TCB

_MESH = plsc.VectorSubcoreMesh(core_axis_name="c", subcore_axis_name="s")


def _degree(dst3, ones16, zeros16):
  @functools.partial(
      pl.kernel,
      out_type=jax.ShapeDtypeStruct((NC, N_PAD, 16), jnp.float32),
      mesh=_MESH,
      scratch_types=[
          pltpu.VMEM((2, CHUNK), jnp.int32),
          pltpu.VMEM((CHUNK, 16), jnp.float32),
          pltpu.VMEM_SHARED((N_PAD, 16), jnp.float32),
          pltpu.SemaphoreType.DMA,
      ],
  )
  def run(dst_hbm, ones_hbm, zeros_hbm, out_hbm, dstr, ones_v, acc, semp):
    c = lax.axis_index("c")
    s = lax.axis_index("s")
    wid = s * NC + c
    base = s * RPT

    pltpu.sync_copy(ones_hbm, ones_v)
    pltpu.sync_copy(zeros_hbm, acc.at[pl.ds(base, RPT)])
    plsc.subcore_barrier()

    pltpu.sync_copy(dst_hbm.at[wid].at[0], dstr.at[0])
    pltpu.async_copy(dst_hbm.at[wid].at[1], dstr.at[1], semp)

    def outer(i, carry):
      j0 = i * 2
      for b in range(2):
        j = j0 + b

        @pl.when(j + 1 < NCHP)
        def _():
          pltpu.make_async_copy(dst_hbm.at[wid].at[j + 1], dstr.at[1 - b], semp).wait()

        pltpu.sync_copy(ones_v, acc.at[dstr.at[b]], add=True)

        @pl.when(j + 2 < NCHP)
        def _():
          pltpu.async_copy(dst_hbm.at[wid].at[j + 2], dstr.at[b], semp)
      return carry

    lax.fori_loop(0, NCHP // 2, outer, 0)
    plsc.subcore_barrier()
    pltpu.sync_copy(acc.at[pl.ds(base, RPT)], out_hbm.at[c].at[pl.ds(base, RPT)])

  return run(dst3, ones16, zeros16)


def _aggregate(mp, src3, dst3, zeros):
  @functools.partial(
      pl.kernel,
      out_type=jax.ShapeDtypeStruct((NC, N_PAD, FEAT), jnp.float32),
      mesh=_MESH,
      scratch_types=[
          pltpu.VMEM((2, CHUNK), jnp.int32),
          pltpu.VMEM((2, CHUNK), jnp.int32),
          pltpu.VMEM((CHUNK, FEAT), jnp.float32),
          pltpu.VMEM((CHUNK, FEAT), jnp.float32),
          pltpu.VMEM_SHARED((N_PAD, FEAT), jnp.float32),
          pltpu.SemaphoreType.DMA,
          pltpu.SemaphoreType.DMA,
      ],
  )
  def run(mp_hbm, src_hbm, dst_hbm, zeros_hbm, out_hbm,
          srcr, dstr, buf0, buf1, acc, sem, semp):
    c = lax.axis_index("c")
    s = lax.axis_index("s")
    wid = s * NC + c
    base = s * RPT

    pltpu.sync_copy(zeros_hbm, acc.at[pl.ds(base, RPT)])
    plsc.subcore_barrier()

    bufs = (buf0, buf1)
    pltpu.sync_copy(src_hbm.at[wid].at[0], srcr.at[0])
    pltpu.sync_copy(dst_hbm.at[wid].at[0], dstr.at[0])

    pltpu.async_copy(mp_hbm.at[srcr.at[0]], buf0, sem)
    pltpu.async_copy(src_hbm.at[wid].at[1], srcr.at[1], semp)
    pltpu.async_copy(dst_hbm.at[wid].at[1], dstr.at[1], semp)

    def outer(i, carry):
      j0 = i * 2
      for b in range(2):
        j = j0 + b

        @pl.when(j + 1 < NCHP)
        def _():
          pltpu.make_async_copy(src_hbm.at[wid].at[j + 1], srcr.at[1 - b], semp).wait()
          pltpu.make_async_copy(dst_hbm.at[wid].at[j + 1], dstr.at[1 - b], semp).wait()

        pltpu.make_async_copy(mp_hbm.at[srcr.at[b]], bufs[b], sem).wait()

        @pl.when(j + 1 < NCHP)
        def _():
          pltpu.async_copy(mp_hbm.at[srcr.at[1 - b]], bufs[1 - b], sem)

        pltpu.sync_copy(bufs[b], acc.at[dstr.at[b]], add=True)

        @pl.when(j + 2 < NCHP)
        def _():
          pltpu.async_copy(src_hbm.at[wid].at[j + 2], srcr.at[b], semp)
          pltpu.async_copy(dst_hbm.at[wid].at[j + 2], dstr.at[b], semp)
      return carry

    lax.fori_loop(0, NCHP // 2, outer, 0)
    plsc.subcore_barrier()
    pltpu.sync_copy(acc.at[pl.ds(base, RPT)], out_hbm.at[c].at[pl.ds(base, RPT)])

  return run(mp, src3, dst3, zeros)


def _tc_first(features, W1, deg):
  def body(f_ref, w_ref, d_ref, mp_ref, dis_ref):
    dis = lax.rsqrt(d_ref[0] + d_ref[1] + 1.0)
    dis_ref[...] = dis
    m = jnp.dot(f_ref[...], w_ref[...],
                preferred_element_type=jnp.float32,
                precision=lax.Precision.HIGHEST)
    mp_ref[...] = m * dis[:, 0:1]

  return pl.pallas_call(
      body,
      grid=(TCG,),
      in_specs=[
          pl.BlockSpec((TCB, FEAT), lambda i: (i, 0)),
          pl.BlockSpec((FEAT, FEAT), lambda i: (0, 0)),
          pl.BlockSpec((NC, TCB, 16), lambda i: (0, i, 0)),
      ],
      out_specs=(pl.BlockSpec((TCB, FEAT), lambda i: (i, 0)),
                 pl.BlockSpec((TCB, 16), lambda i: (i, 0))),
      out_shape=(jax.ShapeDtypeStruct((N_PAD, FEAT), jnp.float32),
                 jax.ShapeDtypeStruct((N_PAD, 16), jnp.float32)),
  )(features, W1, deg)


def _tc_mid(acc, mp, dis16, b, W, feat_out):
  def body(a_ref, mp_ref, dis_ref, b_ref, w_ref, o_ref):
    dis = dis_ref[...][:, 0:1]
    x = jnp.maximum(dis * (a_ref[0] + a_ref[1] + mp_ref[...]) + b_ref[...], 0.0)
    m = jnp.dot(x, w_ref[...],
                preferred_element_type=jnp.float32,
                precision=lax.Precision.HIGHEST) * dis
    if feat_out < FEAT:
      m = jnp.concatenate(
          [m, jnp.zeros((TCB, FEAT - feat_out), jnp.float32)], axis=1)
    o_ref[...] = m

  return pl.pallas_call(
      body,
      grid=(TCG,),
      in_specs=[
          pl.BlockSpec((NC, TCB, FEAT), lambda i: (0, i, 0)),
          pl.BlockSpec((TCB, FEAT), lambda i: (i, 0)),
          pl.BlockSpec((TCB, 16), lambda i: (i, 0)),
          pl.BlockSpec((1, FEAT), lambda i: (0, 0)),
          pl.BlockSpec((FEAT, feat_out), lambda i: (0, 0)),
      ],
      out_specs=pl.BlockSpec((TCB, FEAT), lambda i: (i, 0)),
      out_shape=jax.ShapeDtypeStruct((N_PAD, FEAT), jnp.float32),
  )(acc, mp, dis16, b, W)


def _tc_final(acc, mp, dis16, b):
  def body(a_ref, mp_ref, dis_ref, b_ref, x_ref, ls_ref):
    dis = dis_ref[...][:, 0:1]
    x = dis * (a_ref[0] + a_ref[1] + mp_ref[...])[:, :64] + b_ref[...]
    x_ref[...] = x
    mx = jnp.max(x, axis=-1, keepdims=True)
    lse = jnp.log(jnp.sum(jnp.exp(x - mx), axis=-1, keepdims=True)) + mx
    ls_ref[...] = x - lse

  return pl.pallas_call(
      body,
      grid=(TCG,),
      in_specs=[
          pl.BlockSpec((NC, TCB, FEAT), lambda i: (0, i, 0)),
          pl.BlockSpec((TCB, FEAT), lambda i: (i, 0)),
          pl.BlockSpec((TCB, 16), lambda i: (i, 0)),
          pl.BlockSpec((1, 64), lambda i: (0, 0)),
      ],
      out_specs=(pl.BlockSpec((TCB, 64), lambda i: (i, 0)),
                 pl.BlockSpec((TCB, 64), lambda i: (i, 0))),
      out_shape=(jax.ShapeDtypeStruct((N_PAD, 64), jnp.float32),
                 jax.ShapeDtypeStruct((N_PAD, 64), jnp.float32)),
  )(acc, mp, dis16, b)


def kernel(features, edge_index, W1, b1, W2, b2, W3, b3):
  featp = jnp.pad(features, ((0, N_PAD - N_NODES), (0, 0)))
  ei = edge_index.astype(jnp.int32)
  n_fill = E_PAD - N_EDGES
  fill = jnp.arange(n_fill, dtype=jnp.int32)
  src3 = jnp.concatenate([ei[0], fill % N_NODES]).reshape(NW, NCHP, CHUNK)
  dst3 = jnp.concatenate([ei[1], N_NODES + fill % (N_PAD - N_NODES)]
                         ).reshape(NW, NCHP, CHUNK)
  zeros128 = jnp.zeros((RPT, FEAT), jnp.float32)
  ones16 = jnp.ones((CHUNK, 16), jnp.float32)
  zeros16 = jnp.zeros((RPT, 16), jnp.float32)
  b1r = b1.reshape(1, -1)
  b2r = b2.reshape(1, -1)
  b3r = b3.reshape(1, -1)

  dega = _degree(dst3, ones16, zeros16)
  mp1, dis16 = _tc_first(featp, W1, dega)
  acc1 = _aggregate(mp1, src3, dst3, zeros128)
  mp2 = _tc_mid(acc1, mp1, dis16, b1r, W2, 128)
  acc2 = _aggregate(mp2, src3, dst3, zeros128)
  mp3 = _tc_mid(acc2, mp2, dis16, b2r, W3, 64)
  acc3 = _aggregate(mp3, src3, dst3, zeros128)
  x3, ls = _tc_final(acc3, mp3, dis16, b3r)
  return (x3[:N_NODES], ls[:N_NODES])


# speedup vs baseline: 22.4674x; 1.0501x over previous
import functools

import jax
import jax.numpy as jnp
from jax import lax
from jax.experimental import pallas as pl
from jax.experimental.pallas import tpu as pltpu
from jax.experimental.pallas import tpu_sc as plsc

N_NODES = 10000
N_PAD = 10240
N_EDGES = 320000
FEAT = 128
NC = 2
NS = 16
NW = NC * NS
CHUNK = 128
NCHP = 80
EPT = NCHP * CHUNK
E_PAD = NW * EPT
GARBAGE = N_PAD - 1
RPT = N_PAD // NS
TCB = 1280
TCG = N_PAD // TCB

_MESH = plsc.VectorSubcoreMesh(core_axis_name="c", subcore_axis_name="s")


def _degree(dst3, ones16, zeros16):
  @functools.partial(
      pl.kernel,
      out_type=jax.ShapeDtypeStruct((NC, N_PAD, 16), jnp.float32),
      mesh=_MESH,
      scratch_types=[
          pltpu.VMEM((4, CHUNK), jnp.int32),
          pltpu.VMEM((CHUNK, 16), jnp.float32),
          pltpu.VMEM_SHARED((N_PAD, 16), jnp.float32),
          pltpu.SemaphoreType.DMA,
          pltpu.SemaphoreType.DMA,
      ],
  )
  def run(dst_hbm, ones_hbm, zeros_hbm, out_hbm, dstr, ones_v, acc, semp, sems):
    c = lax.axis_index("c")
    s = lax.axis_index("s")
    wid = s * NC + c
    base = s * RPT

    pltpu.sync_copy(ones_hbm, ones_v)
    pltpu.sync_copy(zeros_hbm, acc.at[pl.ds(base, RPT)])
    plsc.subcore_barrier()

    pltpu.sync_copy(dst_hbm.at[wid].at[0], dstr.at[0])
    pltpu.sync_copy(dst_hbm.at[wid].at[1], dstr.at[1])

    def scat(q):
      pltpu.async_copy(ones_v, acc.at[dstr.at[q]], sems, add=True)

    def scat_wait(q):
      pltpu.make_async_copy(ones_v, acc.at[dstr.at[q]], sems).wait()

    def outer(i, carry):
      j0 = i * 4
      for q in range(4):
        j = j0 + q
        scat(q)

        @pl.when(j >= 2)
        def _():
          scat_wait((q - 2) % 4)

        @pl.when(j + 2 < NCHP)
        def _():
          pltpu.async_copy(dst_hbm.at[wid].at[j + 2], dstr.at[(q + 2) % 4], semp)

        @pl.when((j + 1 >= 2) & (j + 1 < NCHP))
        def _():
          pltpu.make_async_copy(dst_hbm.at[wid].at[j + 1],
                                dstr.at[(q + 1) % 4], semp).wait()
      return carry

    lax.fori_loop(0, NCHP // 4, outer, 0)
    scat_wait((NCHP - 2) % 4)
    scat_wait((NCHP - 1) % 4)
    plsc.subcore_barrier()
    pltpu.sync_copy(acc.at[pl.ds(base, RPT)], out_hbm.at[c].at[pl.ds(base, RPT)])

  return run(dst3, ones16, zeros16)


def _aggregate(mp, src3, dst3, zeros):
  @functools.partial(
      pl.kernel,
      out_type=jax.ShapeDtypeStruct((NC, N_PAD, FEAT), jnp.float32),
      mesh=_MESH,
      scratch_types=[
          pltpu.VMEM((4, CHUNK), jnp.int32),
          pltpu.VMEM((4, CHUNK), jnp.int32),
          pltpu.VMEM((CHUNK, FEAT), jnp.float32),
          pltpu.VMEM((CHUNK, FEAT), jnp.float32),
          pltpu.VMEM_SHARED((N_PAD, FEAT), jnp.float32),
          pltpu.SemaphoreType.DMA,
          pltpu.SemaphoreType.DMA,
          pltpu.SemaphoreType.DMA,
      ],
  )
  def run(mp_hbm, src_hbm, dst_hbm, zeros_hbm, out_hbm,
          srcr, dstr, buf0, buf1, acc, sem, semp, sems):
    c = lax.axis_index("c")
    s = lax.axis_index("s")
    wid = s * NC + c
    base = s * RPT

    pltpu.sync_copy(zeros_hbm, acc.at[pl.ds(base, RPT)])
    plsc.subcore_barrier()

    bufs = (buf0, buf1)

    def idx_start(row, slot):
      pltpu.async_copy(src_hbm.at[wid].at[row], srcr.at[slot], semp)
      pltpu.async_copy(dst_hbm.at[wid].at[row], dstr.at[slot], semp)

    def idx_wait(row, slot):
      pltpu.make_async_copy(src_hbm.at[wid].at[row], srcr.at[slot], semp).wait()
      pltpu.make_async_copy(dst_hbm.at[wid].at[row], dstr.at[slot], semp).wait()

    def scat_wait(slot, b):
      pltpu.make_async_copy(bufs[b], acc.at[dstr.at[slot]], sems).wait()

    pltpu.sync_copy(src_hbm.at[wid].at[0], srcr.at[0])
    pltpu.sync_copy(dst_hbm.at[wid].at[0], dstr.at[0])
    pltpu.sync_copy(src_hbm.at[wid].at[1], srcr.at[1])
    pltpu.sync_copy(dst_hbm.at[wid].at[1], dstr.at[1])
    pltpu.async_copy(mp_hbm.at[srcr.at[0]], buf0, sem)

    def outer(i, carry):
      j0 = i * 4
      for q in range(4):
        j = j0 + q
        b = q % 2

        pltpu.make_async_copy(mp_hbm.at[srcr.at[q]], bufs[b], sem).wait()
        pltpu.async_copy(bufs[b], acc.at[dstr.at[q]], sems, add=True)

        @pl.when(j >= 1)
        def _():
          scat_wait((q - 1) % 4, 1 - b)

        @pl.when(j + 2 < NCHP)
        def _():
          idx_start(j + 2, (q + 2) % 4)

        @pl.when((j + 1 >= 2) & (j + 1 < NCHP))
        def _():
          idx_wait(j + 1, (q + 1) % 4)

        @pl.when(j + 1 < NCHP)
        def _():
          pltpu.async_copy(mp_hbm.at[srcr.at[(q + 1) % 4]], bufs[1 - b], sem)
      return carry

    lax.fori_loop(0, NCHP // 4, outer, 0)
    scat_wait((NCHP - 1) % 4, (NCHP - 1) % 2)
    plsc.subcore_barrier()
    pltpu.sync_copy(acc.at[pl.ds(base, RPT)], out_hbm.at[c].at[pl.ds(base, RPT)])

  return run(mp, src3, dst3, zeros)


def _tc_first(features, W1, deg):
  def body(f_ref, w_ref, d_ref, mp_ref, dis_ref):
    dis = lax.rsqrt(d_ref[0] + d_ref[1] + 1.0)
    dis_ref[...] = dis
    m = jnp.dot(f_ref[...], w_ref[...],
                preferred_element_type=jnp.float32,
                precision=lax.Precision.HIGHEST)
    mp_ref[...] = m * dis[:, 0:1]

  return pl.pallas_call(
      body,
      grid=(TCG,),
      in_specs=[
          pl.BlockSpec((TCB, FEAT), lambda i: (i, 0)),
          pl.BlockSpec((FEAT, FEAT), lambda i: (0, 0)),
          pl.BlockSpec((NC, TCB, 16), lambda i: (0, i, 0)),
      ],
      out_specs=(pl.BlockSpec((TCB, FEAT), lambda i: (i, 0)),
                 pl.BlockSpec((TCB, 16), lambda i: (i, 0))),
      out_shape=(jax.ShapeDtypeStruct((N_PAD, FEAT), jnp.float32),
                 jax.ShapeDtypeStruct((N_PAD, 16), jnp.float32)),
  )(features, W1, deg)


def _tc_mid(acc, mp, dis16, b, W, feat_out):
  def body(a_ref, mp_ref, dis_ref, b_ref, w_ref, o_ref):
    dis = dis_ref[...][:, 0:1]
    x = jnp.maximum(dis * (a_ref[0] + a_ref[1] + mp_ref[...]) + b_ref[...], 0.0)
    m = jnp.dot(x, w_ref[...],
                preferred_element_type=jnp.float32,
                precision=lax.Precision.HIGHEST) * dis
    if feat_out < FEAT:
      m = jnp.concatenate(
          [m, jnp.zeros((TCB, FEAT - feat_out), jnp.float32)], axis=1)
    o_ref[...] = m

  return pl.pallas_call(
      body,
      grid=(TCG,),
      in_specs=[
          pl.BlockSpec((NC, TCB, FEAT), lambda i: (0, i, 0)),
          pl.BlockSpec((TCB, FEAT), lambda i: (i, 0)),
          pl.BlockSpec((TCB, 16), lambda i: (i, 0)),
          pl.BlockSpec((1, FEAT), lambda i: (0, 0)),
          pl.BlockSpec((FEAT, feat_out), lambda i: (0, 0)),
      ],
      out_specs=pl.BlockSpec((TCB, FEAT), lambda i: (i, 0)),
      out_shape=jax.ShapeDtypeStruct((N_PAD, FEAT), jnp.float32),
  )(acc, mp, dis16, b, W)


def _tc_final(acc, mp, dis16, b):
  def body(a_ref, mp_ref, dis_ref, b_ref, x_ref, ls_ref):
    dis = dis_ref[...][:, 0:1]
    x = dis * (a_ref[0] + a_ref[1] + mp_ref[...])[:, :64] + b_ref[...]
    x_ref[...] = x
    mx = jnp.max(x, axis=-1, keepdims=True)
    lse = jnp.log(jnp.sum(jnp.exp(x - mx), axis=-1, keepdims=True)) + mx
    ls_ref[...] = x - lse

  return pl.pallas_call(
      body,
      grid=(TCG,),
      in_specs=[
          pl.BlockSpec((NC, TCB, FEAT), lambda i: (0, i, 0)),
          pl.BlockSpec((TCB, FEAT), lambda i: (i, 0)),
          pl.BlockSpec((TCB, 16), lambda i: (i, 0)),
          pl.BlockSpec((1, 64), lambda i: (0, 0)),
      ],
      out_specs=(pl.BlockSpec((TCB, 64), lambda i: (i, 0)),
                 pl.BlockSpec((TCB, 64), lambda i: (i, 0))),
      out_shape=(jax.ShapeDtypeStruct((N_PAD, 64), jnp.float32),
                 jax.ShapeDtypeStruct((N_PAD, 64), jnp.float32)),
  )(acc, mp, dis16, b)


def kernel(features, edge_index, W1, b1, W2, b2, W3, b3):
  featp = jnp.pad(features, ((0, N_PAD - N_NODES), (0, 0)))
  ei = edge_index.astype(jnp.int32)
  n_fill = E_PAD - N_EDGES
  fill = jnp.arange(n_fill, dtype=jnp.int32)
  src3 = jnp.concatenate([ei[0], fill % N_NODES]).reshape(NW, NCHP, CHUNK)
  dst3 = jnp.concatenate([ei[1], N_NODES + fill % (N_PAD - N_NODES)]
                         ).reshape(NW, NCHP, CHUNK)
  zeros128 = jnp.zeros((RPT, FEAT), jnp.float32)
  ones16 = jnp.ones((CHUNK, 16), jnp.float32)
  zeros16 = jnp.zeros((RPT, 16), jnp.float32)
  b1r = b1.reshape(1, -1)
  b2r = b2.reshape(1, -1)
  b3r = b3.reshape(1, -1)

  dega = _degree(dst3, ones16, zeros16)
  mp1, dis16 = _tc_first(featp, W1, dega)
  acc1 = _aggregate(mp1, src3, dst3, zeros128)
  mp2 = _tc_mid(acc1, mp1, dis16, b1r, W2, 128)
  acc2 = _aggregate(mp2, src3, dst3, zeros128)
  mp3 = _tc_mid(acc2, mp2, dis16, b2r, W3, 64)
  acc3 = _aggregate(mp3, src3, dst3, zeros128)
  x3, ls = _tc_final(acc3, mp3, dis16, b3r)
  return (x3[:N_NODES], ls[:N_NODES])
